# Initial kernel scaffold; baseline (speedup 1.0000x reference)
import jax
import jax.numpy as jnp
from jax.experimental import pallas as pl


def kernel(x, Wl1, bl1, Wr1, Wl2, bl2, Wr2, W1, b1, gamma, beta, W2, b2, edge_index):
    raise NotImplementedError("write your pallas kernel here")


# speedup vs baseline: 2.4360x; 2.4360x over previous
import functools

import jax
import jax.numpy as jnp
from jax import lax
from jax.experimental import pallas as pl
from jax.experimental.pallas import tpu as pltpu
from jax.experimental.pallas import tpu_sc as plsc

N = 40020
E = 640320
F = 116
H = 64
NP = 40960
EPT = 40320
E_PAD = 16 * EPT
B = 128
RPT = NP // 16
D1 = 116 * 64
D2 = 116 * 32
G = 345
GP = 352

_SC_MESH = dict(core_axis_name="c", subcore_axis_name="s",
                num_cores=2, num_subcores=16)


def _softplus(v):
    return jnp.where(v > 20.0, v, jnp.log1p(jnp.exp(jnp.minimum(v, 20.0))))


def _mish(v):
    return v * jnp.tanh(_softplus(v))


@functools.lru_cache(maxsize=None)
def _make_sc_agg(nch):
    cpc = nch // 2

    @functools.partial(
        pl.kernel,
        out_type=jax.ShapeDtypeStruct((nch * NP, 32), jnp.float32),
        mesh=plsc.VectorSubcoreMesh(**_SC_MESH),
        scratch_types=[
            pltpu.VMEM((B,), jnp.int32),
            pltpu.VMEM((B,), jnp.int32),
            pltpu.VMEM((B,), jnp.int32),
            pltpu.VMEM((B, 32), jnp.float32),
            pltpu.VMEM_SHARED((NP, 32), jnp.float32),
            pltpu.SemaphoreType.DMA,
        ],
        compiler_params=pltpu.CompilerParams(use_tc_tiling_on_sc=False),
    )
    def sc_agg(src_hbm, dst_hbm, table_hbm, zeros_hbm, out_hbm,
               srcv, dstv, adjv, rows, acc, sem):
        c = lax.axis_index("c")
        s = lax.axis_index("s")
        row0 = s * RPT
        for ci in range(cpc):
            chunk = c * cpc + ci
            off = chunk * NP
            pltpu.sync_copy(zeros_hbm.at[pl.ds(row0, RPT)],
                            acc.at[pl.ds(row0, RPT)])
            plsc.subcore_barrier()

            base_e = s * EPT

            def step(j, carry):
                eb = base_e + j * B
                pltpu.sync_copy(src_hbm.at[pl.ds(eb, B)], srcv)
                pltpu.sync_copy(dst_hbm.at[pl.ds(eb, B)], dstv)
                for k in range(B // 16):
                    sl = pl.ds(k * 16, 16)
                    adjv[sl] = srcv[sl] + off
                pltpu.async_copy(table_hbm.at[adjv], rows, sem).wait()
                pltpu.sync_copy(rows, acc.at[dstv], add=True)
                return carry

            lax.fori_loop(0, EPT // B, step, 0)
            plsc.subcore_barrier()
            pltpu.sync_copy(acc.at[pl.ds(row0, RPT)],
                            out_hbm.at[pl.ds(off + row0, RPT)])

    return sc_agg


_BN = 256


def _tc1_body(agg_ref, x_ref, wl1_ref, wr1_ref, bl1_ref, w2_ref, b2_ref,
              out_ref):
    a = agg_ref[...]
    deg = a[:, 116:117]
    dn = 1.0 / jnp.maximum(deg, 1.0)
    acc = jnp.dot(a * dn, wl1_ref[...], preferred_element_type=jnp.float32)
    acc = acc + jnp.dot(x_ref[...], wr1_ref[...],
                        preferred_element_type=jnp.float32)
    h1 = _mish(acc + bl1_ref[...])
    out_ref[...] = jnp.dot(h1, w2_ref[...],
                           preferred_element_type=jnp.float32) + b2_ref[...]


def _tc2_body(a2_ref, ps_ref, c3_ref, out_ref):
    deg = c3_ref[:, 20:21]
    dn = 1.0 / jnp.maximum(deg, 1.0)
    out_ref[...] = _mish(a2_ref[...] * dn + ps_ref[:, 64:128])


def _tc3_body(x_ref, w_ref, b_ref, out_ref):
    out_ref[...] = jnp.dot(x_ref[...], w_ref[...],
                           preferred_element_type=jnp.float32) + b_ref[...]


def _tc4_body(h_ref, gamma_ref, beta_ref, w_ref, b_ref, out_ref):
    h = h_ref[...]
    mu = jnp.mean(h, axis=-1, keepdims=True)
    var = jnp.mean((h - mu) ** 2, axis=-1, keepdims=True)
    hn = (h - mu) * lax.rsqrt(var + 1e-5)
    hn = hn * gamma_ref[...] + beta_ref[...]
    hm = _mish(hn)
    out_ref[...] = jnp.dot(hm, w_ref[...],
                           preferred_element_type=jnp.float32) + b_ref[...]


def _chunked(mat, nch):
    return mat.reshape(NP, nch, 32).transpose(1, 0, 2).reshape(nch * NP, 32)


def _unchunk(mat, nch):
    return mat.reshape(nch, NP, 32).transpose(1, 0, 2).reshape(NP, nch * 32)


def kernel(x, Wl1, bl1, Wr1, Wl2, bl2, Wr2, W1, b1, gamma, beta, W2, b2,
           edge_index):
    f32 = jnp.float32
    src = edge_index[0].astype(jnp.int32)
    dst = edge_index[1].astype(jnp.int32)
    pad_e = jnp.full((E_PAD - E,), N, jnp.int32)
    src_p = jnp.concatenate([src, pad_e])
    dst_p = jnp.concatenate([dst, pad_e])

    ones_col = jnp.ones((N, 1), f32)
    x1 = jnp.zeros((NP, 128), f32)
    x1 = x1.at[:N, :116].set(x.astype(f32))
    x1 = x1.at[:N, 116:117].set(ones_col)
    xc = _chunked(x1, 4)

    zeros32 = jnp.zeros((NP, 32), f32)

    agg1c = _make_sc_agg(4)(src_p, dst_p, xc, zeros32)
    agg1n = _unchunk(agg1c, 4)

    wl1 = jnp.zeros((128, 128), f32).at[:116, :116].set(Wl1.T)
    wr1 = jnp.zeros((128, 128), f32).at[:116, :116].set(Wr1.T)
    bl1p = jnp.zeros((1, 128), f32).at[0, :116].set(bl1)
    w2cat = jnp.zeros((128, 128), f32)
    w2cat = w2cat.at[:116, :64].set(Wl2.T).at[:116, 64:128].set(Wr2.T)
    b2cat = jnp.zeros((1, 128), f32).at[0, 64:128].set(bl2)

    nblk = NP // _BN
    row_spec = pl.BlockSpec((_BN, 128), lambda i: (i, 0))
    w_spec = pl.BlockSpec((128, 128), lambda i: (0, 0))
    bias_spec = pl.BlockSpec((1, 128), lambda i: (0, 0))
    ps = pl.pallas_call(
        _tc1_body,
        grid=(nblk,),
        in_specs=[row_spec, row_spec, w_spec, w_spec, bias_spec, w_spec,
                  bias_spec],
        out_specs=row_spec,
        out_shape=jax.ShapeDtypeStruct((NP, 128), f32),
    )(agg1n, x1, wl1, wr1, bl1p, w2cat, b2cat)

    p2c = _chunked(ps[:, :64], 2)
    agg2c = _make_sc_agg(2)(src_p, dst_p, p2c, zeros32)
    agg2n = _unchunk(agg2c, 2)

    a2_spec = pl.BlockSpec((_BN, 64), lambda i: (i, 0))
    c3_spec = pl.BlockSpec((_BN, 32), lambda i: (i, 0))
    h2 = pl.pallas_call(
        _tc2_body,
        grid=(nblk,),
        in_specs=[a2_spec, row_spec, c3_spec],
        out_specs=a2_spec,
        out_shape=jax.ShapeDtypeStruct((NP, 64), f32),
    )(agg2n, ps, lax.slice_in_dim(agg1c, 3 * NP, 4 * NP, axis=0))

    x3 = h2[:N].reshape(G, D1)
    x3 = jnp.pad(x3, ((0, GP - G), (0, 0)))
    w1t = W1.T
    b1r = b1.reshape(1, D2)
    hmid = pl.pallas_call(
        _tc3_body,
        grid=(D2 // 128,),
        in_specs=[
            pl.BlockSpec((GP, D1), lambda j: (0, 0)),
            pl.BlockSpec((D1, 128), lambda j: (0, j)),
            pl.BlockSpec((1, 128), lambda j: (0, j)),
        ],
        out_specs=pl.BlockSpec((GP, 128), lambda j: (0, j)),
        out_shape=jax.ShapeDtypeStruct((GP, D2), f32),
    )(x3, w1t, b1r)

    w2t = jnp.zeros((D2, 128), f32).at[:, :2].set(W2.T)
    b2r = jnp.zeros((1, 128), f32).at[0, :2].set(b2)
    out128 = pl.pallas_call(
        _tc4_body,
        in_specs=[
            pl.BlockSpec((GP, D2), lambda: (0, 0)),
            pl.BlockSpec((1, D2), lambda: (0, 0)),
            pl.BlockSpec((1, D2), lambda: (0, 0)),
            pl.BlockSpec((D2, 128), lambda: (0, 0)),
            pl.BlockSpec((1, 128), lambda: (0, 0)),
        ],
        out_specs=pl.BlockSpec((GP, 128), lambda: (0, 0)),
        out_shape=jax.ShapeDtypeStruct((GP, 128), f32),
    )(hmid, gamma.reshape(1, D2), beta.reshape(1, D2), w2t, b2r)

    return out128[:G, :2]


# speedup vs baseline: 3.7574x; 1.5424x over previous
import functools

import jax
import jax.numpy as jnp
from jax import lax
from jax.experimental import pallas as pl
from jax.experimental.pallas import tpu as pltpu
from jax.experimental.pallas import tpu_sc as plsc

N = 40020
E = 640320
F = 116
H = 64
NP = 40960
B = 128
GB = 4
EPT = 40960
NG = EPT // (GB * B)
E_PAD = 16 * EPT
E_OVER = E_PAD + GB * B
ROWS_PT = EPT // B
RPT = NP // 16
D1 = 116 * 64
D2 = 116 * 32
G = 345
GP = 352

_SC_MESH = dict(core_axis_name="c", subcore_axis_name="s",
                num_cores=2, num_subcores=16)


def _softplus(v):
    return jnp.where(v > 20.0, v, jnp.log1p(jnp.exp(jnp.minimum(v, 20.0))))


def _mish(v):
    return v * jnp.tanh(_softplus(v))


@functools.lru_cache(maxsize=None)
def _make_sc_agg(nch):
    cpc = nch // 2

    @functools.partial(
        pl.kernel,
        out_type=jax.ShapeDtypeStruct((nch * NP, 32), jnp.float32),
        mesh=plsc.VectorSubcoreMesh(**_SC_MESH),
        scratch_types=[
            pltpu.VMEM((2, GB, B), jnp.int32),
            pltpu.VMEM((2, GB, B), jnp.int32),
            pltpu.VMEM((2, GB, B), jnp.int32),
            pltpu.VMEM((2, GB, B, 32), jnp.float32),
            pltpu.VMEM_SHARED((NP, 32), jnp.float32),
            pltpu.SemaphoreType.DMA,
            pltpu.SemaphoreType.DMA,
            pltpu.SemaphoreType.DMA,
            pltpu.SemaphoreType.DMA,
            pltpu.SemaphoreType.DMA,
        ],
        compiler_params=pltpu.CompilerParams(use_tc_tiling_on_sc=False),
    )
    def sc_agg(src_hbm, dst_hbm, table_hbm, zeros_hbm, zdrain_hbm, out_hbm,
               src3, dst3, sdst3, rows, acc,
               gsem, ssem0, ssem1, isem0, isem1):
        c = lax.axis_index("c")
        s = lax.axis_index("s")
        row0 = s * RPT
        base_r = s * ROWS_PT
        ssems = (ssem0, ssem1)
        isems = (isem0, isem1)
        for ci in range(cpc):
            chunk = c * cpc + ci
            off = chunk * NP
            pltpu.sync_copy(zeros_hbm.at[pl.ds(row0, RPT)],
                            acc.at[pl.ds(row0, RPT)])
            plsc.subcore_barrier()
            for b in range(2):
                pltpu.async_copy(src_hbm.at[pl.ds(base_r + b * GB, GB)],
                                 src3.at[b], isems[b])
                pltpu.async_copy(dst_hbm.at[pl.ds(base_r + b * GB, GB)],
                                 dst3.at[b], isems[b])
                pltpu.async_copy(zdrain_hbm, rows.at[b], ssems[b])

            def pair(h, carry):
                for b in range(2):
                    goff = base_r + 2 * h * GB + b * GB
                    pltpu.make_async_copy(zdrain_hbm, rows.at[b],
                                          ssems[b]).wait()
                    pltpu.make_async_copy(src_hbm.at[pl.ds(0, GB)],
                                          src3.at[b], isems[b]).wait()
                    pltpu.make_async_copy(dst_hbm.at[pl.ds(0, GB)],
                                          dst3.at[b], isems[b]).wait()
                    for j in range(GB):
                        for k in range(B // 16):
                            sl = pl.ds(k * 16, 16)
                            src3[b, j, sl] = src3[b, j, sl] + off
                            sdst3[b, j, sl] = dst3[b, j, sl]
                    for j in range(GB):
                        pltpu.async_copy(table_hbm.at[src3.at[b, j]],
                                         rows.at[b, j], gsem)
                    pltpu.make_async_copy(zdrain_hbm, rows.at[b], gsem).wait()
                    pltpu.async_copy(src_hbm.at[pl.ds(goff + 2 * GB, GB)],
                                     src3.at[b], isems[b])
                    pltpu.async_copy(dst_hbm.at[pl.ds(goff + 2 * GB, GB)],
                                     dst3.at[b], isems[b])
                    for j in range(GB):
                        pltpu.async_copy(rows.at[b, j], acc.at[sdst3.at[b, j]],
                                         ssems[b], add=True)
                return carry

            lax.fori_loop(0, NG // 2, pair, 0)
            for b in range(2):
                pltpu.make_async_copy(zdrain_hbm, rows.at[b], ssems[b]).wait()
                pltpu.make_async_copy(src_hbm.at[pl.ds(0, GB)],
                                      src3.at[b], isems[b]).wait()
                pltpu.make_async_copy(dst_hbm.at[pl.ds(0, GB)],
                                      dst3.at[b], isems[b]).wait()
            plsc.subcore_barrier()
            pltpu.sync_copy(acc.at[pl.ds(row0, RPT)],
                            out_hbm.at[pl.ds(off + row0, RPT)])

    return sc_agg


_BN = 256


def _tc1_body(agg_ref, x_ref, wl1_ref, wr1_ref, bl1_ref, w2_ref, b2_ref,
              out_ref):
    a = agg_ref[...]
    deg = a[:, 116:117]
    dn = 1.0 / jnp.maximum(deg, 1.0)
    acc = jnp.dot(a * dn, wl1_ref[...], preferred_element_type=jnp.float32)
    acc = acc + jnp.dot(x_ref[...], wr1_ref[...],
                        preferred_element_type=jnp.float32)
    h1 = _mish(acc + bl1_ref[...])
    out_ref[...] = jnp.dot(h1, w2_ref[...],
                           preferred_element_type=jnp.float32) + b2_ref[...]


def _tc2_body(a2_ref, ps_ref, c3_ref, out_ref):
    deg = c3_ref[:, 20:21]
    dn = 1.0 / jnp.maximum(deg, 1.0)
    out_ref[...] = _mish(a2_ref[...] * dn + ps_ref[:, 64:128])


def _tc3_body(x_ref, w_ref, b_ref, out_ref):
    out_ref[...] = jnp.dot(x_ref[...], w_ref[...],
                           preferred_element_type=jnp.float32) + b_ref[...]


def _tc4_body(h_ref, gamma_ref, beta_ref, w_ref, b_ref, out_ref):
    h = h_ref[...]
    mu = jnp.mean(h, axis=-1, keepdims=True)
    var = jnp.mean((h - mu) ** 2, axis=-1, keepdims=True)
    hn = (h - mu) * lax.rsqrt(var + 1e-5)
    hn = hn * gamma_ref[...] + beta_ref[...]
    hm = _mish(hn)
    out_ref[...] = jnp.dot(hm, w_ref[...],
                           preferred_element_type=jnp.float32) + b_ref[...]


def _chunked(mat, nch):
    return mat.reshape(NP, nch, 32).transpose(1, 0, 2).reshape(nch * NP, 32)


def _unchunk(mat, nch):
    return mat.reshape(nch, NP, 32).transpose(1, 0, 2).reshape(NP, nch * 32)


def kernel(x, Wl1, bl1, Wr1, Wl2, bl2, Wr2, W1, b1, gamma, beta, W2, b2,
           edge_index):
    f32 = jnp.float32
    src = edge_index[0].astype(jnp.int32)
    dst = edge_index[1].astype(jnp.int32)
    pad_e = jnp.full((E_OVER - E,), N, jnp.int32)
    src_p = jnp.concatenate([src, pad_e]).reshape(E_OVER // B, B)
    dst_p = jnp.concatenate([dst, pad_e]).reshape(E_OVER // B, B)
    zdrain = jnp.zeros((GB, B, 32), f32)

    ones_col = jnp.ones((N, 1), f32)
    x1 = jnp.zeros((NP, 128), f32)
    x1 = x1.at[:N, :116].set(x.astype(f32))
    x1 = x1.at[:N, 116:117].set(ones_col)
    xc = _chunked(x1, 4)

    zeros32 = jnp.zeros((NP, 32), f32)

    agg1c = _make_sc_agg(4)(src_p, dst_p, xc, zeros32, zdrain)
    agg1n = _unchunk(agg1c, 4)

    wl1 = jnp.zeros((128, 128), f32).at[:116, :116].set(Wl1.T)
    wr1 = jnp.zeros((128, 128), f32).at[:116, :116].set(Wr1.T)
    bl1p = jnp.zeros((1, 128), f32).at[0, :116].set(bl1)
    w2cat = jnp.zeros((128, 128), f32)
    w2cat = w2cat.at[:116, :64].set(Wl2.T).at[:116, 64:128].set(Wr2.T)
    b2cat = jnp.zeros((1, 128), f32).at[0, 64:128].set(bl2)

    nblk = NP // _BN
    row_spec = pl.BlockSpec((_BN, 128), lambda i: (i, 0))
    w_spec = pl.BlockSpec((128, 128), lambda i: (0, 0))
    bias_spec = pl.BlockSpec((1, 128), lambda i: (0, 0))
    ps = pl.pallas_call(
        _tc1_body,
        grid=(nblk,),
        in_specs=[row_spec, row_spec, w_spec, w_spec, bias_spec, w_spec,
                  bias_spec],
        out_specs=row_spec,
        out_shape=jax.ShapeDtypeStruct((NP, 128), f32),
    )(agg1n, x1, wl1, wr1, bl1p, w2cat, b2cat)

    p2c = _chunked(ps[:, :64], 2)
    agg2c = _make_sc_agg(2)(src_p, dst_p, p2c, zeros32, zdrain)
    agg2n = _unchunk(agg2c, 2)

    a2_spec = pl.BlockSpec((_BN, 64), lambda i: (i, 0))
    c3_spec = pl.BlockSpec((_BN, 32), lambda i: (i, 0))
    h2 = pl.pallas_call(
        _tc2_body,
        grid=(nblk,),
        in_specs=[a2_spec, row_spec, c3_spec],
        out_specs=a2_spec,
        out_shape=jax.ShapeDtypeStruct((NP, 64), f32),
    )(agg2n, ps, lax.slice_in_dim(agg1c, 3 * NP, 4 * NP, axis=0))

    x3 = h2[:N].reshape(G, D1)
    x3 = jnp.pad(x3, ((0, GP - G), (0, 0)))
    w1t = W1.T
    b1r = b1.reshape(1, D2)
    hmid = pl.pallas_call(
        _tc3_body,
        grid=(D2 // 128,),
        in_specs=[
            pl.BlockSpec((GP, D1), lambda j: (0, 0)),
            pl.BlockSpec((D1, 128), lambda j: (0, j)),
            pl.BlockSpec((1, 128), lambda j: (0, j)),
        ],
        out_specs=pl.BlockSpec((GP, 128), lambda j: (0, j)),
        out_shape=jax.ShapeDtypeStruct((GP, D2), f32),
    )(x3, w1t, b1r)

    w2t = jnp.zeros((D2, 128), f32).at[:, :2].set(W2.T)
    b2r = jnp.zeros((1, 128), f32).at[0, :2].set(b2)
    out128 = pl.pallas_call(
        _tc4_body,
        in_specs=[
            pl.BlockSpec((GP, D2), lambda: (0, 0)),
            pl.BlockSpec((1, D2), lambda: (0, 0)),
            pl.BlockSpec((1, D2), lambda: (0, 0)),
            pl.BlockSpec((D2, 128), lambda: (0, 0)),
            pl.BlockSpec((1, 128), lambda: (0, 0)),
        ],
        out_specs=pl.BlockSpec((GP, 128), lambda: (0, 0)),
        out_shape=jax.ShapeDtypeStruct((GP, 128), f32),
    )(hmid, gamma.reshape(1, D2), beta.reshape(1, D2), w2t, b2r)

    return out128[:G, :2]
